# baseline (speedup 1.0000x reference)
import functools

import jax
import jax.numpy as jnp
from jax import lax
from jax.experimental import pallas as pl
from jax.experimental.pallas import tpu as pltpu
from jax.experimental.pallas import tpu_sc as plsc

_N = 10000
_NP = 10240
_E = 160000
_F = 256
_HF = 128
_NC = 2
_NS = 16
_CH = 80
_EPS = _E // _NS
_NCHUNK = _EPS // _CH
_CHC = 40
_EPW = _E // (_NC * _NS)
_NCHUNKC = _EPW // _CHC
_RPS = _NP // _NS
_ZR = 128
_RB = 1000
_NBUF = 3


def _mesh():
    return plsc.VectorSubcoreMesh(
        core_axis_name="c", subcore_axis_name="s",
        num_cores=_NC, num_subcores=_NS)


def _make_seg_sum():
    out_type = (
        jax.ShapeDtypeStruct((_NP, _HF), jnp.float32),
        jax.ShapeDtypeStruct((_NP, _HF), jnp.float32),
    )
    scratch = (
        pltpu.VMEM_SHARED((_NP, _HF), jnp.float32),
        pltpu.VMEM((_ZR, _HF), jnp.float32),
    ) + _NBUF * (
        pltpu.VMEM((_CH,), jnp.int32),
        pltpu.VMEM((_CH,), jnp.int32),
        pltpu.VMEM((_CH, _HF), jnp.float32),
        pltpu.SemaphoreType.DMA,
        pltpu.SemaphoreType.DMA,
    )

    def body(src_h, dst_h, tlo_h, thi_h, z_h, slo_h, shi_h,
             acc, zbuf, *bufrefs):
        cid = lax.axis_index("c")
        sid = lax.axis_index("s")
        r0 = sid * _RPS

        pltpu.sync_copy(z_h, zbuf)
        for j in range(_RPS // _ZR):
            pltpu.sync_copy(zbuf, acc.at[pl.ds(r0 + j * _ZR, _ZR)])
        plsc.subcore_barrier()

        e0 = sid * _EPS
        bufs = tuple(tuple(bufrefs[5 * i:5 * i + 5]) for i in range(_NBUF))

        def run(tab_h):
            def start(b, c):
                sv, dv, rv, sm, _ = b
                off = e0 + c * _CH
                pltpu.sync_copy(src_h.at[pl.ds(off, _CH)], sv)
                pltpu.sync_copy(dst_h.at[pl.ds(off, _CH)], dv)
                pltpu.async_copy(tab_h.at[sv], rv, sm)

            def finish(b):
                sv, dv, rv, sm, ssm = b
                pltpu.make_async_copy(tab_h.at[sv], rv, sm).wait()
                pltpu.async_copy(rv, acc.at[dv], ssm, add=True)

            def drain(b):
                sv, dv, rv, sm, ssm = b
                pltpu.make_async_copy(rv, acc.at[dv], ssm).wait()

            for i in range(_NBUF):
                start(bufs[i], i)

            def bodyn(cn, carry):
                for i in range(_NBUF):
                    b = bufs[i]
                    c = cn * _NBUF + i
                    finish(b)
                    nxt = c + _NBUF

                    @pl.when(nxt < _NCHUNK)
                    def _():
                        drain(b)
                        start(b, nxt)
                return carry
            lax.fori_loop(0, _NCHUNK // _NBUF, bodyn, 0)
            for c in range(_NBUF * (_NCHUNK // _NBUF), _NCHUNK):
                finish(bufs[c % _NBUF])
            for i in range(_NBUF):
                drain(bufs[i])

        @pl.when(cid == 0)
        def _():
            run(tlo_h)

        @pl.when(cid == 1)
        def _():
            run(thi_h)

        plsc.subcore_barrier()

        @pl.when(cid == 0)
        def _():
            pltpu.sync_copy(acc.at[pl.ds(r0, _RPS)], slo_h.at[pl.ds(r0, _RPS)])

        @pl.when(cid == 1)
        def _():
            pltpu.sync_copy(acc.at[pl.ds(r0, _RPS)], shi_h.at[pl.ds(r0, _RPS)])

    return pl.kernel(body, out_type=out_type, mesh=_mesh(),
                     scratch_types=scratch)


def _make_count():
    out_type = (
        jax.ShapeDtypeStruct((_NP, _HF), jnp.float32),
        jax.ShapeDtypeStruct((_NP, _HF), jnp.float32),
    )
    scratch = (
        pltpu.VMEM_SHARED((_NP, _HF), jnp.float32),
        pltpu.VMEM((_ZR, _HF), jnp.float32),
        pltpu.VMEM((_CHC, _HF), jnp.float32),
    ) + _NBUF * (
        pltpu.VMEM((_CHC,), jnp.int32),
        pltpu.SemaphoreType.DMA,
    )

    def body(dst_h, z_h, o_h, p0_h, p1_h, acc, zbuf, ones_v, *bufrefs):
        cid = lax.axis_index("c")
        sid = lax.axis_index("s")
        r0 = sid * _RPS

        pltpu.sync_copy(z_h, zbuf)
        for j in range(_RPS // _ZR):
            pltpu.sync_copy(zbuf, acc.at[pl.ds(r0 + j * _ZR, _ZR)])
        pltpu.sync_copy(o_h, ones_v)
        plsc.subcore_barrier()

        e0 = (cid * _NS + sid) * _EPW
        bufs = tuple(tuple(bufrefs[2 * i:2 * i + 2]) for i in range(_NBUF))

        def chunkop(b, c):
            dv, sm = b
            off = e0 + c * _CHC
            pltpu.sync_copy(dst_h.at[pl.ds(off, _CHC)], dv)
            pltpu.async_copy(ones_v, acc.at[dv], sm, add=True)

        def drain(b):
            dv, sm = b
            pltpu.make_async_copy(ones_v, acc.at[dv], sm).wait()

        for i in range(_NBUF):
            chunkop(bufs[i], i)

        def bodyn(cn, carry):
            for i in range(_NBUF):
                b = bufs[i]
                c = cn * _NBUF + i
                nxt = c + _NBUF

                @pl.when(nxt < _NCHUNKC)
                def _():
                    drain(b)
                    chunkop(b, nxt)
            return carry
        lax.fori_loop(0, _NCHUNKC // _NBUF + 1, bodyn, 0)
        for i in range(_NBUF):
            drain(bufs[i])

        plsc.subcore_barrier()

        @pl.when(cid == 0)
        def _():
            pltpu.sync_copy(acc.at[pl.ds(r0, _RPS)], p0_h.at[pl.ds(r0, _RPS)])

        @pl.when(cid == 1)
        def _():
            pltpu.sync_copy(acc.at[pl.ds(r0, _RPS)], p1_h.at[pl.ds(r0, _RPS)])

    return pl.kernel(body, out_type=out_type, mesh=_mesh(),
                     scratch_types=scratch)


def _dense_pre_body(xl, xh, wr, b, olo, ohi):
    xx = jnp.concatenate([xl[...], xh[...]], axis=1)
    y = jnp.dot(xx, wr[...], preferred_element_type=jnp.float32) + b[0:1, :]
    olo[...] = y[:, :_HF]
    ohi[...] = y[:, _HF:]


def _dense_pre(xl, xh, wr, b):
    rowspec = pl.BlockSpec((_RB, _HF), lambda i: (i, 0))
    return pl.pallas_call(
        _dense_pre_body,
        grid=(_N // _RB,),
        in_specs=[rowspec, rowspec,
                  pl.BlockSpec((_F, _F), lambda i: (0, 0)),
                  pl.BlockSpec((8, _F), lambda i: (0, 0))],
        out_specs=[rowspec, rowspec],
        out_shape=[jax.ShapeDtypeStruct((_N, _HF), jnp.float32)] * 2,
    )(xl, xh, wr, b)


def _dense_post_body(relu, slo, shi, p0, p1, rl, rh, wl, olo, ohi):
    c = jnp.maximum(p0[:, 0:1] + p1[:, 0:1], 1.0)
    s = jnp.concatenate([slo[...], shi[...]], axis=1)
    r = jnp.concatenate([rl[...], rh[...]], axis=1)
    y = jnp.dot(s / c, wl[...], preferred_element_type=jnp.float32) + r
    if relu:
        y = jnp.maximum(y, 0.0)
    olo[...] = y[:, :_HF]
    ohi[...] = y[:, _HF:]


def _dense_post(relu, slo, shi, p0, p1, rl, rh, wl):
    rowspec = pl.BlockSpec((_RB, _HF), lambda i: (i, 0))
    return pl.pallas_call(
        functools.partial(_dense_post_body, relu),
        grid=(_N // _RB,),
        in_specs=[rowspec, rowspec, rowspec, rowspec, rowspec, rowspec,
                  pl.BlockSpec((_F, _F), lambda i: (0, 0))],
        out_specs=[rowspec, rowspec],
        out_shape=[jax.ShapeDtypeStruct((_N, _HF), jnp.float32)] * 2,
    )(slo, shi, p0, p1, rl, rh, wl)


def kernel(x, edge_index, W1_l, b1_l, W1_r,
           Wmu_l, bmu_l, Wmu_r, Wls_l, bls_l, Wls_r):
    src = edge_index[0].astype(jnp.int32)
    dst = edge_index[1].astype(jnp.int32)
    xlo = x[:, :_HF]
    xhi = x[:, _HF:]
    z = jnp.zeros((_ZR, _HF), jnp.float32)
    ones = jnp.ones((_CHC, _HF), jnp.float32)

    seg = _make_seg_sum()
    count = _make_count()

    b1 = jnp.tile(b1_l[None, :], (8, 1))
    wl2 = jnp.concatenate([Wmu_l.T, Wls_l.T], axis=1)
    wr2 = jnp.concatenate([Wmu_r.T, Wls_r.T], axis=1)
    b2 = jnp.tile(jnp.concatenate([bmu_l, bls_l])[None, :], (8, 1))

    p0, p1 = count(dst, z, ones)
    xr_lo, xr_hi = _dense_pre(xlo, xhi, W1_r.T, b1)
    slo, shi = seg(src, dst, xlo, xhi, z)
    hlo, hhi = _dense_post(True, slo, shi, p0, p1, xr_lo, xr_hi, W1_l.T)

    hr_lo, hr_hi = _dense_pre(hlo, hhi, wr2, b2)
    slo2, shi2 = seg(src, dst, hlo, hhi, z)
    mu, ls = _dense_post(False, slo2, shi2, p0, p1, hr_lo, hr_hi, wl2)
    return (mu, ls)


# speedup vs baseline: 1.3444x; 1.3444x over previous
import functools

import jax
import jax.numpy as jnp
from jax import lax
from jax.experimental import pallas as pl
from jax.experimental.pallas import tpu as pltpu
from jax.experimental.pallas import tpu_sc as plsc

_N = 10000
_NP = 10240
_E = 160000
_F = 256
_HF = 128
_NC = 2
_NS = 16
_CH = 40
_EPS = _E // _NS
_NCHUNK = _EPS // _CH
_CHC = 40
_EPW = _E // (_NC * _NS)
_NCHUNKC = _EPW // _CHC
_RPS = _NP // _NS
_ZR = 64
_RB = 1000
_NBUF = 3


def _mesh():
    return plsc.VectorSubcoreMesh(
        core_axis_name="c", subcore_axis_name="s",
        num_cores=_NC, num_subcores=_NS)


def _make_seg_sum():
    out_type = (
        jax.ShapeDtypeStruct((_NP, _HF), jnp.float32),
        jax.ShapeDtypeStruct((_NP, _HF), jnp.float32),
    )
    scratch = (
        (pltpu.VMEM_SHARED((_NP, _HF), jnp.float32),)
        + (pltpu.VMEM((_ZR, _HF), jnp.float32),)
        + 6 * (pltpu.VMEM((_CH,), jnp.int32),)
        + 12 * (pltpu.VMEM((_CH,), jnp.int32),)
        + 6 * (pltpu.VMEM((_CH, _HF), jnp.float32),)
        + 18 * (pltpu.SemaphoreType.DMA,)
    )

    def body(src_h, dst_h, tlo_h, thi_h, z_h, slo_h, shi_h,
             acc, zbuf, *bufrefs):
        cid = lax.axis_index("c")
        sid = lax.axis_index("s")
        r0 = sid * _RPS

        sv = bufrefs[0:6]
        dv = bufrefs[6:18]
        rv = bufrefs[18:24]
        isem = bufrefs[24:30]
        gsem = bufrefs[30:36]
        ssem = bufrefs[36:42]

        pltpu.sync_copy(z_h, zbuf)
        for j in range(_RPS // _ZR):
            pltpu.sync_copy(zbuf, acc.at[pl.ds(r0 + j * _ZR, _ZR)])
        plsc.subcore_barrier()

        e0 = sid * _EPS

        def run(tab_h):
            def idx_start(c, i, j):
                off = e0 + c * _CH
                pltpu.async_copy(src_h.at[pl.ds(off, _CH)], sv[i], isem[i])
                pltpu.async_copy(dst_h.at[pl.ds(off, _CH)], dv[j], isem[i])

            def idx_wait(c, i, j):
                off = e0 + c * _CH
                pltpu.make_async_copy(
                    src_h.at[pl.ds(off, _CH)], sv[i], isem[i]).wait()
                pltpu.make_async_copy(
                    dst_h.at[pl.ds(off, _CH)], dv[j], isem[i]).wait()

            def gather_start(i):
                pltpu.async_copy(tab_h.at[sv[i]], rv[i], gsem[i])

            def finish(i, j):
                pltpu.make_async_copy(tab_h.at[sv[i]], rv[i], gsem[i]).wait()
                pltpu.async_copy(rv[i], acc.at[dv[j]], ssem[i], add=True)

            def sdrain(i, j):
                pltpu.make_async_copy(rv[i], acc.at[dv[j]], ssem[i]).wait()

            def slot(c, it=None):
                cm6, cm12 = c % 6, c % 12
                cc = c if it is None else it * 12 + cm12
                finish(cm6, cm12)
                if c + 6 < _NCHUNK or it is not None:
                    idx_start(cc + 6, (c + 6) % 6, (c + 6) % 12)
                if c + 3 < _NCHUNK or it is not None:
                    if c >= 3 or it is not None:
                        sdrain((c + 3) % 6, (c + 3) % 12)
                    idx_wait(cc + 3, (c + 3) % 6, (c + 3) % 12)
                    gather_start((c + 3) % 6)

            for c in range(6):
                idx_start(c, c % 6, c % 12)
            for c in range(3):
                idx_wait(c, c % 6, c % 12)
                gather_start(c % 6)
            for c in range(12):
                slot(c)

            def body12(it, carry):
                for i in range(12):
                    slot(12 + i, it=it)
                return carry
            lax.fori_loop(1, _NCHUNK // 12, body12, 0)

            for c in range(12 * (_NCHUNK // 12), _NCHUNK):
                slot(c)
            for c in range(_NCHUNK - 6, _NCHUNK):
                sdrain(c % 6, c % 12)

        @pl.when(cid == 0)
        def _():
            run(tlo_h)

        @pl.when(cid == 1)
        def _():
            run(thi_h)

        plsc.subcore_barrier()

        @pl.when(cid == 0)
        def _():
            pltpu.sync_copy(acc.at[pl.ds(r0, _RPS)], slo_h.at[pl.ds(r0, _RPS)])

        @pl.when(cid == 1)
        def _():
            pltpu.sync_copy(acc.at[pl.ds(r0, _RPS)], shi_h.at[pl.ds(r0, _RPS)])

    return pl.kernel(body, out_type=out_type, mesh=_mesh(),
                     scratch_types=scratch)


def _make_count():
    out_type = (
        jax.ShapeDtypeStruct((_NP, _HF), jnp.float32),
        jax.ShapeDtypeStruct((_NP, _HF), jnp.float32),
    )
    scratch = (
        pltpu.VMEM_SHARED((_NP, _HF), jnp.float32),
        pltpu.VMEM((_ZR, _HF), jnp.float32),
        pltpu.VMEM((_CHC, _HF), jnp.float32),
    ) + _NBUF * (
        pltpu.VMEM((_CHC,), jnp.int32),
        pltpu.SemaphoreType.DMA,
    )

    def body(dst_h, z_h, o_h, p0_h, p1_h, acc, zbuf, ones_v, *bufrefs):
        cid = lax.axis_index("c")
        sid = lax.axis_index("s")
        r0 = sid * _RPS

        pltpu.sync_copy(z_h, zbuf)
        for j in range(_RPS // _ZR):
            pltpu.sync_copy(zbuf, acc.at[pl.ds(r0 + j * _ZR, _ZR)])
        pltpu.sync_copy(o_h, ones_v)
        plsc.subcore_barrier()

        e0 = (cid * _NS + sid) * _EPW
        bufs = tuple(tuple(bufrefs[2 * i:2 * i + 2]) for i in range(_NBUF))

        def chunkop(b, c):
            dv, sm = b
            off = e0 + c * _CHC
            pltpu.sync_copy(dst_h.at[pl.ds(off, _CHC)], dv)
            pltpu.async_copy(ones_v, acc.at[dv], sm, add=True)

        def drain(b):
            dv, sm = b
            pltpu.make_async_copy(ones_v, acc.at[dv], sm).wait()

        for i in range(_NBUF):
            chunkop(bufs[i], i)

        def bodyn(cn, carry):
            for i in range(_NBUF):
                b = bufs[i]
                c = cn * _NBUF + i
                nxt = c + _NBUF

                @pl.when(nxt < _NCHUNKC)
                def _():
                    drain(b)
                    chunkop(b, nxt)
            return carry
        lax.fori_loop(0, _NCHUNKC // _NBUF + 1, bodyn, 0)
        for i in range(_NBUF):
            drain(bufs[i])

        plsc.subcore_barrier()

        @pl.when(cid == 0)
        def _():
            pltpu.sync_copy(acc.at[pl.ds(r0, _RPS)], p0_h.at[pl.ds(r0, _RPS)])

        @pl.when(cid == 1)
        def _():
            pltpu.sync_copy(acc.at[pl.ds(r0, _RPS)], p1_h.at[pl.ds(r0, _RPS)])

    return pl.kernel(body, out_type=out_type, mesh=_mesh(),
                     scratch_types=scratch)


def _dense_pre_body(xl, xh, wr, b, olo, ohi):
    xx = jnp.concatenate([xl[...], xh[...]], axis=1)
    y = jnp.dot(xx, wr[...], preferred_element_type=jnp.float32) + b[0:1, :]
    olo[...] = y[:, :_HF]
    ohi[...] = y[:, _HF:]


def _dense_pre(xl, xh, wr, b):
    rowspec = pl.BlockSpec((_RB, _HF), lambda i: (i, 0))
    return pl.pallas_call(
        _dense_pre_body,
        grid=(_N // _RB,),
        in_specs=[rowspec, rowspec,
                  pl.BlockSpec((_F, _F), lambda i: (0, 0)),
                  pl.BlockSpec((8, _F), lambda i: (0, 0))],
        out_specs=[rowspec, rowspec],
        out_shape=[jax.ShapeDtypeStruct((_N, _HF), jnp.float32)] * 2,
    )(xl, xh, wr, b)


def _dense_post_body(relu, slo, shi, p0, p1, rl, rh, wl, olo, ohi):
    c = jnp.maximum(p0[:, 0:1] + p1[:, 0:1], 1.0)
    s = jnp.concatenate([slo[...], shi[...]], axis=1)
    r = jnp.concatenate([rl[...], rh[...]], axis=1)
    y = jnp.dot(s / c, wl[...], preferred_element_type=jnp.float32) + r
    if relu:
        y = jnp.maximum(y, 0.0)
    olo[...] = y[:, :_HF]
    ohi[...] = y[:, _HF:]


def _dense_post(relu, slo, shi, p0, p1, rl, rh, wl):
    rowspec = pl.BlockSpec((_RB, _HF), lambda i: (i, 0))
    return pl.pallas_call(
        functools.partial(_dense_post_body, relu),
        grid=(_N // _RB,),
        in_specs=[rowspec, rowspec, rowspec, rowspec, rowspec, rowspec,
                  pl.BlockSpec((_F, _F), lambda i: (0, 0))],
        out_specs=[rowspec, rowspec],
        out_shape=[jax.ShapeDtypeStruct((_N, _HF), jnp.float32)] * 2,
    )(slo, shi, p0, p1, rl, rh, wl)


def kernel(x, edge_index, W1_l, b1_l, W1_r,
           Wmu_l, bmu_l, Wmu_r, Wls_l, bls_l, Wls_r):
    src = edge_index[0].astype(jnp.int32)
    dst = edge_index[1].astype(jnp.int32)
    xlo = x[:, :_HF]
    xhi = x[:, _HF:]
    z = jnp.zeros((_ZR, _HF), jnp.float32)
    ones = jnp.ones((_CHC, _HF), jnp.float32)

    seg = _make_seg_sum()
    count = _make_count()

    b1 = jnp.tile(b1_l[None, :], (8, 1))
    wl2 = jnp.concatenate([Wmu_l.T, Wls_l.T], axis=1)
    wr2 = jnp.concatenate([Wmu_r.T, Wls_r.T], axis=1)
    b2 = jnp.tile(jnp.concatenate([bmu_l, bls_l])[None, :], (8, 1))

    p0, p1 = count(dst, z, ones)
    xr_lo, xr_hi = _dense_pre(xlo, xhi, W1_r.T, b1)
    slo, shi = seg(src, dst, xlo, xhi, z)
    hlo, hhi = _dense_post(True, slo, shi, p0, p1, xr_lo, xr_hi, W1_l.T)

    hr_lo, hr_hi = _dense_pre(hlo, hhi, wr2, b2)
    slo2, shi2 = seg(src, dst, hlo, hhi, z)
    mu, ls = _dense_post(False, slo2, shi2, p0, p1, hr_lo, hr_hi, wl2)
    return (mu, ls)


# speedup vs baseline: 1.4380x; 1.0696x over previous
import functools

import jax
import jax.numpy as jnp
from jax import lax
from jax.experimental import pallas as pl
from jax.experimental.pallas import tpu as pltpu
from jax.experimental.pallas import tpu_sc as plsc

_N = 10000
_NP = 10240
_E = 160000
_F = 256
_HF = 128
_NC = 2
_NS = 16
_CH = 40
_EPS = _E // _NS
_NCHUNK = _EPS // _CH
_CHC = 40
_EPW = _E // (_NC * _NS)
_NCHUNKC = _EPW // _CHC
_RPS = _NP // _NS
_ZR = 64
_RB = 1000
_NBUF = 3


def _mesh():
    return plsc.VectorSubcoreMesh(
        core_axis_name="c", subcore_axis_name="s",
        num_cores=_NC, num_subcores=_NS)


def _make_seg_sum():
    out_type = (
        jax.ShapeDtypeStruct((_NP, _HF), jnp.float32),
        jax.ShapeDtypeStruct((_NP, _HF), jnp.float32),
    )
    scratch = (
        (pltpu.VMEM_SHARED((_NP, _HF), jnp.float32),)
        + (pltpu.VMEM((_ZR, _HF), jnp.float32),)
        + 6 * (pltpu.VMEM((_CH,), jnp.int32),)
        + 12 * (pltpu.VMEM((_CH,), jnp.int32),)
        + 6 * (pltpu.VMEM((_CH, _HF), jnp.float32),)
        + 18 * (pltpu.SemaphoreType.DMA,)
    )

    def body(src_h, dst_h, tlo_h, thi_h, z_h, slo_h, shi_h,
             acc, zbuf, *bufrefs):
        cid = lax.axis_index("c")
        sid = lax.axis_index("s")
        r0 = sid * _RPS

        sv = bufrefs[0:6]
        dv = bufrefs[6:18]
        rv = bufrefs[18:24]
        isem = bufrefs[24:30]
        gsem = bufrefs[30:36]
        ssem = bufrefs[36:42]

        pltpu.sync_copy(z_h, zbuf)
        for j in range(_RPS // _ZR):
            pltpu.sync_copy(zbuf, acc.at[pl.ds(r0 + j * _ZR, _ZR)])
        plsc.subcore_barrier()

        e0 = sid * _EPS

        def run(tab_h):
            def idx_start(c, i, j):
                off = e0 + c * _CH
                pltpu.async_copy(src_h.at[pl.ds(off, _CH)], sv[i], isem[i])
                pltpu.async_copy(dst_h.at[pl.ds(off, _CH)], dv[j], isem[i])

            def idx_wait(c, i, j):
                off = e0 + c * _CH
                pltpu.make_async_copy(
                    src_h.at[pl.ds(off, _CH)], sv[i], isem[i]).wait()
                pltpu.make_async_copy(
                    dst_h.at[pl.ds(off, _CH)], dv[j], isem[i]).wait()

            def gather_start(i):
                pltpu.async_copy(tab_h.at[sv[i]], rv[i], gsem[i])

            def finish(i, j):
                pltpu.make_async_copy(tab_h.at[sv[i]], rv[i], gsem[i]).wait()
                pltpu.async_copy(rv[i], acc.at[dv[j]], ssem[i], add=True)

            def sdrain(i, j):
                pltpu.make_async_copy(rv[i], acc.at[dv[j]], ssem[i]).wait()

            def slot(c, it=None):
                cm6, cm12 = c % 6, c % 12
                cc = c if it is None else it * 12 + cm12
                finish(cm6, cm12)
                if c + 6 < _NCHUNK or it is not None:
                    idx_start(cc + 6, (c + 6) % 6, (c + 6) % 12)
                if c + 3 < _NCHUNK or it is not None:
                    if c >= 3 or it is not None:
                        sdrain((c + 3) % 6, (c + 3) % 12)
                    idx_wait(cc + 3, (c + 3) % 6, (c + 3) % 12)
                    gather_start((c + 3) % 6)

            for c in range(6):
                idx_start(c, c % 6, c % 12)
            for c in range(3):
                idx_wait(c, c % 6, c % 12)
                gather_start(c % 6)
            for c in range(12):
                slot(c)

            def body12(it, carry):
                for i in range(12):
                    slot(12 + i, it=it)
                return carry
            lax.fori_loop(1, _NCHUNK // 12, body12, 0)

            for c in range(12 * (_NCHUNK // 12), _NCHUNK):
                slot(c)
            for c in range(_NCHUNK - 6, _NCHUNK):
                sdrain(c % 6, c % 12)

        @pl.when(cid == 0)
        def _():
            run(tlo_h)

        @pl.when(cid == 1)
        def _():
            run(thi_h)

        plsc.subcore_barrier()

        @pl.when(cid == 0)
        def _():
            pltpu.sync_copy(acc.at[pl.ds(r0, _RPS)], slo_h.at[pl.ds(r0, _RPS)])

        @pl.when(cid == 1)
        def _():
            pltpu.sync_copy(acc.at[pl.ds(r0, _RPS)], shi_h.at[pl.ds(r0, _RPS)])

    return pl.kernel(body, out_type=out_type, mesh=_mesh(),
                     scratch_types=scratch)


def _make_count():
    out_type = (
        jax.ShapeDtypeStruct((_NP, _HF), jnp.float32),
        jax.ShapeDtypeStruct((_NP, _HF), jnp.float32),
    )
    scratch = (
        (pltpu.VMEM_SHARED((_NP, _HF), jnp.float32),)
        + (pltpu.VMEM((_ZR, _HF), jnp.float32),)
        + (pltpu.VMEM((_CHC, _HF), jnp.float32),)
        + 12 * (pltpu.VMEM((_CHC,), jnp.int32),)
        + 24 * (pltpu.SemaphoreType.DMA,)
    )

    def body(dst_h, z_h, o_h, p0_h, p1_h, acc, zbuf, ones_v, *bufrefs):
        cid = lax.axis_index("c")
        sid = lax.axis_index("s")
        r0 = sid * _RPS

        dv = bufrefs[0:12]
        isem = bufrefs[12:24]
        ssem = bufrefs[24:36]

        pltpu.sync_copy(z_h, zbuf)
        for j in range(_RPS // _ZR):
            pltpu.sync_copy(zbuf, acc.at[pl.ds(r0 + j * _ZR, _ZR)])
        pltpu.sync_copy(o_h, ones_v)
        plsc.subcore_barrier()

        e0 = (cid * _NS + sid) * _EPW

        def idx_start(c, j):
            off = e0 + c * _CHC
            pltpu.async_copy(dst_h.at[pl.ds(off, _CHC)], dv[j], isem[j])

        def idx_wait(c, j):
            off = e0 + c * _CHC
            pltpu.make_async_copy(
                dst_h.at[pl.ds(off, _CHC)], dv[j], isem[j]).wait()

        def sdrain(j):
            pltpu.make_async_copy(ones_v, acc.at[dv[j]], ssem[j]).wait()

        def slot(c, it=None):
            cm12 = c % 12
            cc = c if it is None else it * 12 + cm12
            idx_wait(cc, cm12)
            pltpu.async_copy(ones_v, acc.at[dv[cm12]], ssem[cm12], add=True)
            if c >= 6 or it is not None:
                sdrain((c + 6) % 12)
            if it is None:
                if c + 6 < _NCHUNKC:
                    idx_start(c + 6, (c + 6) % 12)
            else:
                @pl.when(cc + 6 < _NCHUNKC)
                def _():
                    idx_start(cc + 6, (c + 6) % 12)

        for c in range(6):
            idx_start(c, c % 12)
        for c in range(12):
            slot(c)

        def body12(it, carry):
            for i in range(12):
                slot(12 + i, it=it)
            return carry
        lax.fori_loop(1, _NCHUNKC // 12, body12, 0)

        for c in range(12 * (_NCHUNKC // 12), _NCHUNKC):
            slot(c)
        for c in range(_NCHUNKC - 6, _NCHUNKC):
            sdrain(c % 12)

        plsc.subcore_barrier()

        @pl.when(cid == 0)
        def _():
            pltpu.sync_copy(acc.at[pl.ds(r0, _RPS)], p0_h.at[pl.ds(r0, _RPS)])

        @pl.when(cid == 1)
        def _():
            pltpu.sync_copy(acc.at[pl.ds(r0, _RPS)], p1_h.at[pl.ds(r0, _RPS)])

    return pl.kernel(body, out_type=out_type, mesh=_mesh(),
                     scratch_types=scratch)


def _dense_pre_body(xl, xh, wr, b, olo, ohi):
    xx = jnp.concatenate([xl[...], xh[...]], axis=1)
    y = jnp.dot(xx, wr[...], preferred_element_type=jnp.float32) + b[0:1, :]
    olo[...] = y[:, :_HF]
    ohi[...] = y[:, _HF:]


def _dense_pre(xl, xh, wr, b):
    rowspec = pl.BlockSpec((_RB, _HF), lambda i: (i, 0))
    return pl.pallas_call(
        _dense_pre_body,
        grid=(_N // _RB,),
        in_specs=[rowspec, rowspec,
                  pl.BlockSpec((_F, _F), lambda i: (0, 0)),
                  pl.BlockSpec((8, _F), lambda i: (0, 0))],
        out_specs=[rowspec, rowspec],
        out_shape=[jax.ShapeDtypeStruct((_N, _HF), jnp.float32)] * 2,
    )(xl, xh, wr, b)


def _dense_post_body(relu, slo, shi, p0, p1, rl, rh, wl, olo, ohi):
    c = jnp.maximum(p0[:, 0:1] + p1[:, 0:1], 1.0)
    s = jnp.concatenate([slo[...], shi[...]], axis=1)
    r = jnp.concatenate([rl[...], rh[...]], axis=1)
    y = jnp.dot(s / c, wl[...], preferred_element_type=jnp.float32) + r
    if relu:
        y = jnp.maximum(y, 0.0)
    olo[...] = y[:, :_HF]
    ohi[...] = y[:, _HF:]


def _dense_post(relu, slo, shi, p0, p1, rl, rh, wl):
    rowspec = pl.BlockSpec((_RB, _HF), lambda i: (i, 0))
    return pl.pallas_call(
        functools.partial(_dense_post_body, relu),
        grid=(_N // _RB,),
        in_specs=[rowspec, rowspec, rowspec, rowspec, rowspec, rowspec,
                  pl.BlockSpec((_F, _F), lambda i: (0, 0))],
        out_specs=[rowspec, rowspec],
        out_shape=[jax.ShapeDtypeStruct((_N, _HF), jnp.float32)] * 2,
    )(slo, shi, p0, p1, rl, rh, wl)


def kernel(x, edge_index, W1_l, b1_l, W1_r,
           Wmu_l, bmu_l, Wmu_r, Wls_l, bls_l, Wls_r):
    src = edge_index[0].astype(jnp.int32)
    dst = edge_index[1].astype(jnp.int32)
    xlo = x[:, :_HF]
    xhi = x[:, _HF:]
    z = jnp.zeros((_ZR, _HF), jnp.float32)
    ones = jnp.ones((_CHC, _HF), jnp.float32)

    seg = _make_seg_sum()
    count = _make_count()

    b1 = jnp.tile(b1_l[None, :], (8, 1))
    wl2 = jnp.concatenate([Wmu_l.T, Wls_l.T], axis=1)
    wr2 = jnp.concatenate([Wmu_r.T, Wls_r.T], axis=1)
    b2 = jnp.tile(jnp.concatenate([bmu_l, bls_l])[None, :], (8, 1))

    p0, p1 = count(dst, z, ones)
    xr_lo, xr_hi = _dense_pre(xlo, xhi, W1_r.T, b1)
    slo, shi = seg(src, dst, xlo, xhi, z)
    hlo, hhi = _dense_post(True, slo, shi, p0, p1, xr_lo, xr_hi, W1_l.T)

    hr_lo, hr_hi = _dense_pre(hlo, hhi, wr2, b2)
    slo2, shi2 = seg(src, dst, hlo, hhi, z)
    mu, ls = _dense_post(False, slo2, shi2, p0, p1, hr_lo, hr_hi, wl2)
    return (mu, ls)


# speedup vs baseline: 1.4695x; 1.0219x over previous
import functools

import jax
import jax.numpy as jnp
from jax import lax
from jax.experimental import pallas as pl
from jax.experimental.pallas import tpu as pltpu
from jax.experimental.pallas import tpu_sc as plsc

_N = 10000
_NP = 10240
_E = 160000
_F = 256
_HF = 128
_NC = 2
_NS = 16
_CH = 40
_EPS = _E // _NS
_NCHUNK = _EPS // _CH
_CHC = 40
_EPW = _E // (_NC * _NS)
_NCHUNKC = _EPW // _CHC
_RPS = _NP // _NS
_ZR = 64
_RB = 1000
_NBUF = 3


def _mesh():
    return plsc.VectorSubcoreMesh(
        core_axis_name="c", subcore_axis_name="s",
        num_cores=_NC, num_subcores=_NS)


def _make_seg_sum():
    out_type = (
        jax.ShapeDtypeStruct((_NP, _HF), jnp.float32),
        jax.ShapeDtypeStruct((_NP, _HF), jnp.float32),
    )
    scratch = (
        (pltpu.VMEM_SHARED((_NP, _HF), jnp.float32),)
        + (pltpu.VMEM((_ZR, _HF), jnp.float32),)
        + 6 * (pltpu.VMEM((_CH,), jnp.int32),)
        + 12 * (pltpu.VMEM((_CH,), jnp.int32),)
        + 6 * (pltpu.VMEM((_CH, _HF), jnp.float32),)
        + 18 * (pltpu.SemaphoreType.DMA,)
    )

    def body(src_h, dst_h, tlo_h, thi_h, z_h, slo_h, shi_h,
             acc, zbuf, *bufrefs):
        cid = lax.axis_index("c")
        sid = lax.axis_index("s")
        r0 = sid * _RPS

        sv = bufrefs[0:6]
        dv = bufrefs[6:18]
        rv = bufrefs[18:24]
        isem = bufrefs[24:30]
        gsem = bufrefs[30:36]
        ssem = bufrefs[36:42]

        pltpu.sync_copy(z_h, zbuf)
        for j in range(_RPS // _ZR):
            pltpu.sync_copy(zbuf, acc.at[pl.ds(r0 + j * _ZR, _ZR)])
        plsc.subcore_barrier()

        e0 = sid * _EPS

        def run(tab_h):
            def idx_start(c, i, j):
                off = e0 + c * _CH
                pltpu.async_copy(src_h.at[pl.ds(off, _CH)], sv[i], isem[i])
                pltpu.async_copy(dst_h.at[pl.ds(off, _CH)], dv[j], isem[i])

            def idx_wait(c, i, j):
                off = e0 + c * _CH
                pltpu.make_async_copy(
                    src_h.at[pl.ds(off, _CH)], sv[i], isem[i]).wait()
                pltpu.make_async_copy(
                    dst_h.at[pl.ds(off, _CH)], dv[j], isem[i]).wait()

            def gather_start(i):
                pltpu.async_copy(tab_h.at[sv[i]], rv[i], gsem[i])

            def finish(i, j):
                pltpu.make_async_copy(tab_h.at[sv[i]], rv[i], gsem[i]).wait()
                pltpu.async_copy(rv[i], acc.at[dv[j]], ssem[i], add=True)

            def sdrain(i, j):
                pltpu.make_async_copy(rv[i], acc.at[dv[j]], ssem[i]).wait()

            def slot(c, it=None):
                cm6, cm12 = c % 6, c % 12
                cc = c if it is None else it * 12 + cm12
                finish(cm6, cm12)
                if c + 6 < _NCHUNK or it is not None:
                    idx_start(cc + 6, (c + 6) % 6, (c + 6) % 12)
                if c + 3 < _NCHUNK or it is not None:
                    if c >= 3 or it is not None:
                        sdrain((c + 3) % 6, (c + 3) % 12)
                    idx_wait(cc + 3, (c + 3) % 6, (c + 3) % 12)
                    gather_start((c + 3) % 6)

            for c in range(6):
                idx_start(c, c % 6, c % 12)
            for c in range(3):
                idx_wait(c, c % 6, c % 12)
                gather_start(c % 6)
            for c in range(12):
                slot(c)

            def body12(it, carry):
                for i in range(12):
                    slot(12 + i, it=it)
                return carry
            lax.fori_loop(1, _NCHUNK // 12, body12, 0)

            for c in range(12 * (_NCHUNK // 12), _NCHUNK):
                slot(c)
            for c in range(_NCHUNK - 6, _NCHUNK):
                sdrain(c % 6, c % 12)

        @pl.when(cid == 0)
        def _():
            run(tlo_h)

        @pl.when(cid == 1)
        def _():
            run(thi_h)

        plsc.subcore_barrier()

        @pl.when(cid == 0)
        def _():
            pltpu.sync_copy(acc.at[pl.ds(r0, _RPS)], slo_h.at[pl.ds(r0, _RPS)])

        @pl.when(cid == 1)
        def _():
            pltpu.sync_copy(acc.at[pl.ds(r0, _RPS)], shi_h.at[pl.ds(r0, _RPS)])

    return pl.kernel(body, out_type=out_type, mesh=_mesh(),
                     scratch_types=scratch)


def _make_count():
    out_type = (
        jax.ShapeDtypeStruct((_NP, _HF), jnp.float32),
        jax.ShapeDtypeStruct((_NP, _HF), jnp.float32),
    )
    scratch = (
        (pltpu.VMEM_SHARED((_NP, _HF), jnp.float32),)
        + (pltpu.VMEM((_ZR, _HF), jnp.float32),)
        + (pltpu.VMEM((_CHC, _HF), jnp.float32),)
        + 12 * (pltpu.VMEM((_CHC,), jnp.int32),)
        + 24 * (pltpu.SemaphoreType.DMA,)
    )

    def body(dst_h, z_h, o_h, p0_h, p1_h, acc, zbuf, ones_v, *bufrefs):
        cid = lax.axis_index("c")
        sid = lax.axis_index("s")
        r0 = sid * _RPS

        dv = bufrefs[0:12]
        isem = bufrefs[12:24]
        ssem = bufrefs[24:36]

        pltpu.sync_copy(z_h, zbuf)
        for j in range(_RPS // _ZR):
            pltpu.sync_copy(zbuf, acc.at[pl.ds(r0 + j * _ZR, _ZR)])
        pltpu.sync_copy(o_h, ones_v)
        plsc.subcore_barrier()

        e0 = (cid * _NS + sid) * _EPW

        def idx_start(c, j):
            off = e0 + c * _CHC
            pltpu.async_copy(dst_h.at[pl.ds(off, _CHC)], dv[j], isem[j])

        def idx_wait(c, j):
            off = e0 + c * _CHC
            pltpu.make_async_copy(
                dst_h.at[pl.ds(off, _CHC)], dv[j], isem[j]).wait()

        def sdrain(j):
            pltpu.make_async_copy(ones_v, acc.at[dv[j]], ssem[j]).wait()

        def slot(c, it=None):
            cm12 = c % 12
            cc = c if it is None else it * 12 + cm12
            idx_wait(cc, cm12)
            pltpu.async_copy(ones_v, acc.at[dv[cm12]], ssem[cm12], add=True)
            if c >= 6 or it is not None:
                sdrain((c + 6) % 12)
            if it is None:
                if c + 6 < _NCHUNKC:
                    idx_start(c + 6, (c + 6) % 12)
            else:
                @pl.when(cc + 6 < _NCHUNKC)
                def _():
                    idx_start(cc + 6, (c + 6) % 12)

        for c in range(6):
            idx_start(c, c % 12)
        for c in range(12):
            slot(c)

        def body12(it, carry):
            for i in range(12):
                slot(12 + i, it=it)
            return carry
        lax.fori_loop(1, _NCHUNKC // 12, body12, 0)

        for c in range(12 * (_NCHUNKC // 12), _NCHUNKC):
            slot(c)
        for c in range(_NCHUNKC - 6, _NCHUNKC):
            sdrain(c % 12)

        plsc.subcore_barrier()

        @pl.when(cid == 0)
        def _():
            pltpu.sync_copy(acc.at[pl.ds(r0, _RPS)], p0_h.at[pl.ds(r0, _RPS)])

        @pl.when(cid == 1)
        def _():
            pltpu.sync_copy(acc.at[pl.ds(r0, _RPS)], p1_h.at[pl.ds(r0, _RPS)])

    return pl.kernel(body, out_type=out_type, mesh=_mesh(),
                     scratch_types=scratch)


def _dense_body(relu, slo, shi, pc, xl, xh, wl, wr, b, olo, ohi):
    c = jnp.maximum(pc[:, 0:1], 1.0)
    ss = jnp.concatenate([slo[...], shi[...]], axis=1)
    xx = jnp.concatenate([xl[...], xh[...]], axis=1)
    y = jnp.dot(ss / c, wl[...], preferred_element_type=jnp.float32)
    y = y + jnp.dot(xx, wr[...], preferred_element_type=jnp.float32)
    y = y + b[0:1, :]
    if relu:
        y = jnp.maximum(y, 0.0)
    olo[...] = y[:, :_HF]
    ohi[...] = y[:, _HF:]


def _dense(relu, slo, shi, pc, xl, xh, wl, wr, b):
    rowspec = pl.BlockSpec((_RB, _HF), lambda i: (i, 0))
    return pl.pallas_call(
        functools.partial(_dense_body, relu),
        grid=(_N // _RB,),
        in_specs=[rowspec, rowspec,
                  pl.BlockSpec((_RB, 8), lambda i: (i, 0)),
                  rowspec, rowspec,
                  pl.BlockSpec((_F, _F), lambda i: (0, 0)),
                  pl.BlockSpec((_F, _F), lambda i: (0, 0)),
                  pl.BlockSpec((8, _F), lambda i: (0, 0))],
        out_specs=[rowspec, rowspec],
        out_shape=[jax.ShapeDtypeStruct((_N, _HF), jnp.float32)] * 2,
    )(slo, shi, pc, xl, xh, wl, wr, b)


def kernel(x, edge_index, W1_l, b1_l, W1_r,
           Wmu_l, bmu_l, Wmu_r, Wls_l, bls_l, Wls_r):
    src = edge_index[0].astype(jnp.int32)
    dst = edge_index[1].astype(jnp.int32)
    xlo = x[:, :_HF]
    xhi = x[:, _HF:]
    z = jnp.zeros((_ZR, _HF), jnp.float32)
    ones = jnp.ones((_CHC, _HF), jnp.float32)

    seg = _make_seg_sum()
    count = _make_count()

    b1 = jnp.tile(b1_l[None, :], (8, 1))
    wl2 = jnp.concatenate([Wmu_l.T, Wls_l.T], axis=1)
    wr2 = jnp.concatenate([Wmu_r.T, Wls_r.T], axis=1)
    b2 = jnp.tile(jnp.concatenate([bmu_l, bls_l])[None, :], (8, 1))

    p0, p1 = count(dst, z, ones)
    pc = p0[:, :8] + p1[:, :8]
    slo, shi = seg(src, dst, xlo, xhi, z)
    hlo, hhi = _dense(True, slo, shi, pc, xlo, xhi, W1_l.T, W1_r.T, b1)

    slo2, shi2 = seg(src, dst, hlo, hhi, z)
    mu, ls = _dense(False, slo2, shi2, pc, hlo, hhi, wl2, wr2, b2)
    return (mu, ls)
